# baseline (speedup 1.0000x reference)
import functools

import jax
import jax.numpy as jnp
from jax import lax
from jax.experimental import pallas as pl
from jax.experimental.pallas import tpu as pltpu
from jax.experimental.pallas import tpu_sc as plsc

_NC = 2
_NS = 16
_NW = _NC * _NS
_L = 16
_K = 128
_KC = 256
_BR = 1024


def _sc_aggregate(Np, D, Ep, trash_row, with_counts):
  Dh = D // _NC
  G = Ep // (_NS * _KC)
  IT = G // 2
  rpt = Np // _NS

  mesh = plsc.VectorSubcoreMesh(core_axis_name="c", subcore_axis_name="s")
  if with_counts:
    out_type = (jax.ShapeDtypeStruct((_NC, Np, Dh), jnp.float32),
                jax.ShapeDtypeStruct((_NC, Np, _L), jnp.float32))
  else:
    out_type = jax.ShapeDtypeStruct((_NC, Np, Dh), jnp.float32)
  nz = rpt // _K
  assert rpt % _K == 0 and G % 2 == 0

  scratch = [
      pltpu.VMEM((G, _KC), jnp.int32),
      pltpu.VMEM((G, _KC), jnp.int32),
      pltpu.VMEM((2, _KC, Dh), jnp.float32),
      pltpu.VMEM_SHARED((Np, Dh), jnp.float32),
      pltpu.SemaphoreType.DMA,
      pltpu.SemaphoreType.DMA,
  ]
  if with_counts:
    scratch += [
        pltpu.VMEM((_KC, _L), jnp.float32),
        pltpu.VMEM((_K, _L), jnp.float32),
        pltpu.VMEM_SHARED((Np, _L), jnp.float32),
    ]

  def body(*refs):
    if with_counts:
      (table, src, dst, zacc, zcnt, ones, acc_out, cnt_out,
       src_a, dst_a, rows, sh_acc, gs0, gs1,
       ones_v, cnt_v, sh_cnt) = refs
    else:
      (table, src, dst, zacc, acc_out,
       src_a, dst_a, rows, sh_acc, gs0, gs1) = refs

    c = lax.axis_index("c")
    s = lax.axis_index("s")
    r0 = s * rpt

    pltpu.sync_copy(src.at[s], src_a)
    pltpu.sync_copy(dst.at[s], dst_a)

    def prep(i, carry):
      for j in range(_KC // _L):
        sl = pl.ds(j * _L, _L)
        sv = src_a[i, sl]
        dv = dst_a[i, sl]
        dst_a[i, sl] = jnp.where(
            sv == dv, jnp.full((_L,), trash_row, jnp.int32), dv)
        src_a[i, sl] = sv * 2 + c
      return carry

    lax.fori_loop(0, G, prep, 0)

    pltpu.sync_copy(zacc, rows.at[0, pl.ds(0, _K)])
    for z in range(nz):
      pltpu.sync_copy(rows.at[0, pl.ds(0, _K)],
                      sh_acc.at[pl.ds(r0 + z * _K, _K)])
    if with_counts:
      pltpu.sync_copy(ones, ones_v)
      pltpu.sync_copy(zcnt, cnt_v)
      for z in range(nz):
        pltpu.sync_copy(cnt_v, sh_cnt.at[pl.ds(r0 + z * _K, _K)])

    plsc.subcore_barrier()

    def g_wait(slot, sem):
      pltpu.make_async_copy(table.at[src_a.at[0]], rows.at[slot], sem).wait()

    def scatter(i, slot):
      pltpu.sync_copy(rows.at[slot], sh_acc.at[dst_a.at[i]], add=True)

    pltpu.async_copy(table.at[src_a.at[0]], rows.at[0], gs0)

    def step(i, carry):
      pltpu.async_copy(table.at[src_a.at[2 * i + 1]], rows.at[1], gs1)
      g_wait(0, gs0)
      scatter(2 * i, 0)
      if with_counts:
        @pl.when(c == 0)
        def _():
          pltpu.sync_copy(ones_v, sh_cnt.at[dst_a.at[2 * i]], add=True)

      @pl.when(i + 1 < IT)
      def _():
        pltpu.async_copy(table.at[src_a.at[2 * i + 2]], rows.at[0], gs0)

      g_wait(1, gs1)
      scatter(2 * i + 1, 1)
      if with_counts:
        @pl.when(c == 1)
        def _():
          pltpu.sync_copy(ones_v, sh_cnt.at[dst_a.at[2 * i + 1]], add=True)
      return carry

    lax.fori_loop(0, IT, step, 0)
    plsc.subcore_barrier()

    for z in range(nz):
      pltpu.sync_copy(sh_acc.at[pl.ds(r0 + z * _K, _K)], rows.at[0, pl.ds(0, _K)])
      pltpu.sync_copy(rows.at[0, pl.ds(0, _K)],
                      acc_out.at[c, pl.ds(r0 + z * _K, _K)])
    if with_counts:
      for z in range(nz):
        pltpu.sync_copy(sh_cnt.at[pl.ds(r0 + z * _K, _K)], cnt_v)
        pltpu.sync_copy(cnt_v, cnt_out.at[c, pl.ds(r0 + z * _K, _K)])

  return pl.kernel(
      body, out_type=out_type, mesh=mesh, scratch_types=scratch,
      compiler_params=pltpu.CompilerParams(use_tc_tiling_on_sc=False))


def _mm_relu_body(x_ref, w_ref, b_ref, o_ref):
  o_ref[...] = jnp.maximum(
      jnp.dot(x_ref[...], w_ref[...], preferred_element_type=jnp.float32)
      + b_ref[...], 0.0)


def _layer0_body(a0, a1, c0, c1, xin, wla, wlb, bl, wa, ba, h_ref, o_ref):
  cnt = jnp.maximum(c0[:, :1] + c1[:, :1], 1.0)
  mean = jnp.concatenate([a0[...], a1[...]], axis=1) / cnt
  h = (jnp.dot(xin[...], wla[...], preferred_element_type=jnp.float32)
       + jnp.dot(mean, wlb[...], preferred_element_type=jnp.float32)
       + bl[...])
  h = jnp.maximum(h, 0.0)
  nrm = jnp.sqrt(jnp.sum(h * h, axis=1, keepdims=True))
  h = h / jnp.maximum(nrm, 1e-12)
  h_ref[...] = h
  o_ref[...] = jnp.maximum(
      jnp.dot(h, wa[...], preferred_element_type=jnp.float32) + ba[...], 0.0)


def _layer1_body(a0, a1, c0, c1, hin, wla, wlb, bl, wp1, bp1, wp2, bp2, o_ref):
  cnt = jnp.maximum(c0[:, :1] + c1[:, :1], 1.0)
  mean = jnp.concatenate([a0[...], a1[...]], axis=1) / cnt
  h = (jnp.dot(hin[...], wla[...], preferred_element_type=jnp.float32)
       + jnp.dot(mean, wlb[...], preferred_element_type=jnp.float32)
       + bl[...])
  h = jnp.maximum(h, 0.0)
  nrm = jnp.sqrt(jnp.sum(h * h, axis=1, keepdims=True))
  h = h / jnp.maximum(nrm, 1e-12)
  p = (jnp.dot(jnp.dot(h, wp1[...], preferred_element_type=jnp.float32)
               + bp1[...], wp2[...], preferred_element_type=jnp.float32)
       + bp2[...])
  m = jnp.max(p, axis=1, keepdims=True)
  sh = p - m
  o_ref[...] = sh - jnp.log(jnp.sum(jnp.exp(sh), axis=1, keepdims=True))


def _row_spec(d):
  return pl.BlockSpec((_BR, d), lambda i: (i, 0))


def _full_spec(r, d):
  return pl.BlockSpec((r, d), lambda i: (0, 0))


def kernel(x, edge_index, batch, Wa0, ba0, Wl0, bl0, Wa1, ba1, Wl1, bl1,
           Wp1, bp1, Wp2, bp2):
  del batch
  N, D = x.shape
  E = edge_index.shape[1]
  H = Wl0.shape[1]
  O = Wp2.shape[1]
  f32 = jnp.float32

  Np = -(-(N + 1) // _BR) * _BR
  G = 2 * -(-E // (2 * _NS * _KC))
  Ep = G * _NS * _KC
  grid = (Np // _BR,)
  Dh = D // _NC

  x_p = jnp.pad(x, ((0, Np - N), (0, 0)))
  src_p = jnp.pad(edge_index[0], (0, Ep - E)).reshape(_NS, G, _KC)
  dst_p = jnp.pad(edge_index[1], (0, Ep - E)).reshape(_NS, G, _KC)
  zacc = jnp.zeros((_K, Dh), f32)
  zcnt = jnp.zeros((_K, _L), f32)
  ones = jnp.ones((_KC, _L), f32)

  out0 = pl.pallas_call(
      _mm_relu_body,
      grid=grid,
      in_specs=[_row_spec(D), _full_spec(D, D), _full_spec(1, D)],
      out_specs=_row_spec(D),
      out_shape=jax.ShapeDtypeStruct((Np, D), f32),
  )(x_p, Wa0, ba0.reshape(1, D))

  acc0, cnt = _sc_aggregate(Np, D, Ep, N, True)(
      out0.reshape(_NC * Np, Dh), src_p, dst_p, zacc, zcnt, ones)

  h0, out1 = pl.pallas_call(
      _layer0_body,
      grid=grid,
      in_specs=[_row_spec(Dh), _row_spec(Dh), _row_spec(_L), _row_spec(_L),
                _row_spec(D), _full_spec(D, H), _full_spec(D, H),
                _full_spec(1, H), _full_spec(H, H), _full_spec(1, H)],
      out_specs=[_row_spec(H), _row_spec(H)],
      out_shape=[jax.ShapeDtypeStruct((Np, H), f32),
                 jax.ShapeDtypeStruct((Np, H), f32)],
  )(acc0[0], acc0[1], cnt[0], cnt[1], x_p, Wl0[:D], Wl0[D:],
    bl0.reshape(1, H), Wa1, ba1.reshape(1, H))

  acc1 = _sc_aggregate(Np, H, Ep, N, False)(
      out1.reshape(_NC * Np, Dh), src_p, dst_p, zacc)

  res = pl.pallas_call(
      _layer1_body,
      grid=grid,
      in_specs=[_row_spec(Dh), _row_spec(Dh), _row_spec(_L), _row_spec(_L),
                _row_spec(H), _full_spec(H, H), _full_spec(H, H),
                _full_spec(1, H), _full_spec(H, H), _full_spec(1, H),
                _full_spec(H, O), _full_spec(1, O)],
      out_specs=_row_spec(O),
      out_shape=jax.ShapeDtypeStruct((Np, O), f32),
  )(acc1[0], acc1[1], cnt[0], cnt[1], h0, Wl1[:D], Wl1[D:],
    bl1.reshape(1, H), Wp1, bp1.reshape(1, H), Wp2, bp2.reshape(1, O))

  return res[:N]


# speedup vs baseline: 1.4167x; 1.4167x over previous
import functools

import jax
import jax.numpy as jnp
from jax import lax
from jax.experimental import pallas as pl
from jax.experimental.pallas import tpu as pltpu
from jax.experimental.pallas import tpu_sc as plsc

_NC = 2
_NS = 16
_NW = _NC * _NS
_L = 16
_K = 128
_KC = 128
_BR = 1024


def _sc_aggregate(Np, D, Ep, trash_row, with_counts):
  Dh = D // _NC
  G = Ep // (_NS * _KC)
  IT = G // 2
  rpt = Np // _NS

  mesh = plsc.VectorSubcoreMesh(core_axis_name="c", subcore_axis_name="s")
  if with_counts:
    out_type = (jax.ShapeDtypeStruct((_NC, Np, Dh), jnp.float32),
                jax.ShapeDtypeStruct((_NC, Np, _L), jnp.float32))
  else:
    out_type = jax.ShapeDtypeStruct((_NC, Np, Dh), jnp.float32)
  nz = rpt // _K
  assert rpt % _K == 0 and G % 2 == 0

  scratch = [
      pltpu.VMEM((G, _KC), jnp.int32),
      pltpu.VMEM((G, _KC), jnp.int32),
      pltpu.VMEM((2, _KC, Dh), jnp.float32),
      pltpu.VMEM_SHARED((Np, Dh), jnp.float32),
      pltpu.SemaphoreType.DMA,
      pltpu.SemaphoreType.DMA,
  ]
  if with_counts:
    scratch += [
        pltpu.VMEM((_KC, _L), jnp.float32),
        pltpu.VMEM((_K, _L), jnp.float32),
        pltpu.VMEM_SHARED((Np, _L), jnp.float32),
    ]

  def body(*refs):
    if with_counts:
      (table, src, dst, zacc, zcnt, ones, acc_out, cnt_out,
       src_a, dst_a, rows, sh_acc, gs0, gs1,
       ones_v, cnt_v, sh_cnt) = refs
    else:
      (table, src, dst, zacc, acc_out,
       src_a, dst_a, rows, sh_acc, gs0, gs1) = refs

    c = lax.axis_index("c")
    s = lax.axis_index("s")
    r0 = s * rpt

    pltpu.sync_copy(src.at[s], src_a)
    pltpu.sync_copy(dst.at[s], dst_a)

    def prep(i, carry):
      for j in range(_KC // _L):
        sl = pl.ds(j * _L, _L)
        sv = src_a[i, sl]
        dv = dst_a[i, sl]
        dst_a[i, sl] = jnp.where(
            sv == dv, jnp.full((_L,), trash_row, jnp.int32), dv)
        src_a[i, sl] = sv * 2 + c
      return carry

    lax.fori_loop(0, G, prep, 0)

    pltpu.sync_copy(zacc, rows.at[0, pl.ds(0, _K)])
    for z in range(nz):
      pltpu.sync_copy(rows.at[0, pl.ds(0, _K)],
                      sh_acc.at[pl.ds(r0 + z * _K, _K)])
    if with_counts:
      pltpu.sync_copy(ones, ones_v)
      pltpu.sync_copy(zcnt, cnt_v)
      for z in range(nz):
        pltpu.sync_copy(cnt_v, sh_cnt.at[pl.ds(r0 + z * _K, _K)])

    plsc.subcore_barrier()

    def g_wait(slot, sem):
      pltpu.make_async_copy(table.at[src_a.at[0]], rows.at[slot], sem).wait()

    def scatter(i, slot):
      pltpu.sync_copy(rows.at[slot], sh_acc.at[dst_a.at[i]], add=True)

    pltpu.async_copy(table.at[src_a.at[0]], rows.at[0], gs0)

    def step(i, carry):
      pltpu.async_copy(table.at[src_a.at[2 * i + 1]], rows.at[1], gs1)
      g_wait(0, gs0)
      scatter(2 * i, 0)
      if with_counts:
        @pl.when(c == 0)
        def _():
          pltpu.sync_copy(ones_v, sh_cnt.at[dst_a.at[2 * i]], add=True)

      @pl.when(i + 1 < IT)
      def _():
        pltpu.async_copy(table.at[src_a.at[2 * i + 2]], rows.at[0], gs0)

      g_wait(1, gs1)
      scatter(2 * i + 1, 1)
      if with_counts:
        @pl.when(c == 1)
        def _():
          pltpu.sync_copy(ones_v, sh_cnt.at[dst_a.at[2 * i + 1]], add=True)
      return carry

    lax.fori_loop(0, IT, step, 0)
    plsc.subcore_barrier()

    for z in range(nz):
      pltpu.sync_copy(sh_acc.at[pl.ds(r0 + z * _K, _K)], rows.at[0, pl.ds(0, _K)])
      pltpu.sync_copy(rows.at[0, pl.ds(0, _K)],
                      acc_out.at[c, pl.ds(r0 + z * _K, _K)])
    if with_counts:
      for z in range(nz):
        pltpu.sync_copy(sh_cnt.at[pl.ds(r0 + z * _K, _K)], cnt_v)
        pltpu.sync_copy(cnt_v, cnt_out.at[c, pl.ds(r0 + z * _K, _K)])

  return pl.kernel(
      body, out_type=out_type, mesh=mesh, scratch_types=scratch,
      compiler_params=pltpu.CompilerParams(use_tc_tiling_on_sc=False))


def _mm_relu_body(x_ref, w_ref, b_ref, o_ref):
  o_ref[...] = jnp.maximum(
      jnp.dot(x_ref[...], w_ref[...], preferred_element_type=jnp.float32)
      + b_ref[...], 0.0)


def _layer0_body(a0, a1, c0, c1, xin, wla, wlb, bl, wa, ba, h_ref, o_ref):
  cnt = jnp.maximum(c0[:, :1] + c1[:, :1], 1.0)
  mean = jnp.concatenate([a0[...], a1[...]], axis=1) / cnt
  h = (jnp.dot(xin[...], wla[...], preferred_element_type=jnp.float32)
       + jnp.dot(mean, wlb[...], preferred_element_type=jnp.float32)
       + bl[...])
  h = jnp.maximum(h, 0.0)
  nrm = jnp.sqrt(jnp.sum(h * h, axis=1, keepdims=True))
  h = h / jnp.maximum(nrm, 1e-12)
  h_ref[...] = h
  o_ref[...] = jnp.maximum(
      jnp.dot(h, wa[...], preferred_element_type=jnp.float32) + ba[...], 0.0)


def _layer1_body(a0, a1, c0, c1, hin, wla, wlb, bl, wp1, bp1, wp2, bp2, o_ref):
  cnt = jnp.maximum(c0[:, :1] + c1[:, :1], 1.0)
  mean = jnp.concatenate([a0[...], a1[...]], axis=1) / cnt
  h = (jnp.dot(hin[...], wla[...], preferred_element_type=jnp.float32)
       + jnp.dot(mean, wlb[...], preferred_element_type=jnp.float32)
       + bl[...])
  h = jnp.maximum(h, 0.0)
  nrm = jnp.sqrt(jnp.sum(h * h, axis=1, keepdims=True))
  h = h / jnp.maximum(nrm, 1e-12)
  p = (jnp.dot(jnp.dot(h, wp1[...], preferred_element_type=jnp.float32)
               + bp1[...], wp2[...], preferred_element_type=jnp.float32)
       + bp2[...])
  m = jnp.max(p, axis=1, keepdims=True)
  sh = p - m
  o_ref[...] = sh - jnp.log(jnp.sum(jnp.exp(sh), axis=1, keepdims=True))


def _row_spec(d):
  return pl.BlockSpec((_BR, d), lambda i: (i, 0))


def _full_spec(r, d):
  return pl.BlockSpec((r, d), lambda i: (0, 0))


def kernel(x, edge_index, batch, Wa0, ba0, Wl0, bl0, Wa1, ba1, Wl1, bl1,
           Wp1, bp1, Wp2, bp2):
  del batch
  N, D = x.shape
  E = edge_index.shape[1]
  H = Wl0.shape[1]
  O = Wp2.shape[1]
  f32 = jnp.float32

  Np = -(-(N + 1) // _BR) * _BR
  G = 2 * -(-E // (2 * _NS * _KC))
  Ep = G * _NS * _KC
  grid = (Np // _BR,)
  Dh = D // _NC

  x_p = jnp.pad(x, ((0, Np - N), (0, 0)))
  src_p = jnp.pad(edge_index[0], (0, Ep - E)).reshape(_NS, G, _KC)
  dst_p = jnp.pad(edge_index[1], (0, Ep - E)).reshape(_NS, G, _KC)
  zacc = jnp.zeros((_K, Dh), f32)
  zcnt = jnp.zeros((_K, _L), f32)
  ones = jnp.ones((_KC, _L), f32)

  out0 = pl.pallas_call(
      _mm_relu_body,
      grid=grid,
      in_specs=[_row_spec(D), _full_spec(D, D), _full_spec(1, D)],
      out_specs=_row_spec(D),
      out_shape=jax.ShapeDtypeStruct((Np, D), f32),
  )(x_p, Wa0, ba0.reshape(1, D))

  acc0, cnt = _sc_aggregate(Np, D, Ep, N, True)(
      out0.reshape(_NC * Np, Dh), src_p, dst_p, zacc, zcnt, ones)

  h0, out1 = pl.pallas_call(
      _layer0_body,
      grid=grid,
      in_specs=[_row_spec(Dh), _row_spec(Dh), _row_spec(_L), _row_spec(_L),
                _row_spec(D), _full_spec(D, H), _full_spec(D, H),
                _full_spec(1, H), _full_spec(H, H), _full_spec(1, H)],
      out_specs=[_row_spec(H), _row_spec(H)],
      out_shape=[jax.ShapeDtypeStruct((Np, H), f32),
                 jax.ShapeDtypeStruct((Np, H), f32)],
  )(acc0[0], acc0[1], cnt[0], cnt[1], x_p, Wl0[:D], Wl0[D:],
    bl0.reshape(1, H), Wa1, ba1.reshape(1, H))

  acc1 = _sc_aggregate(Np, H, Ep, N, False)(
      out1.reshape(_NC * Np, Dh), src_p, dst_p, zacc)

  res = pl.pallas_call(
      _layer1_body,
      grid=grid,
      in_specs=[_row_spec(Dh), _row_spec(Dh), _row_spec(_L), _row_spec(_L),
                _row_spec(H), _full_spec(H, H), _full_spec(H, H),
                _full_spec(1, H), _full_spec(H, H), _full_spec(1, H),
                _full_spec(H, O), _full_spec(1, O)],
      out_specs=_row_spec(O),
      out_shape=jax.ShapeDtypeStruct((Np, O), f32),
  )(acc1[0], acc1[1], cnt[0], cnt[1], h0, Wl1[:D], Wl1[D:],
    bl1.reshape(1, H), Wp1, bp1.reshape(1, H), Wp2, bp2.reshape(1, O))

  return res[:N]


# speedup vs baseline: 1.4501x; 1.0236x over previous
import functools

import jax
import jax.numpy as jnp
from jax import lax
from jax.experimental import pallas as pl
from jax.experimental.pallas import tpu as pltpu
from jax.experimental.pallas import tpu_sc as plsc

_NC = 2
_NS = 16
_NW = _NC * _NS
_L = 16
_K = 128
_KC = 128
_BR = 1024


def _sc_aggregate(Np, D, Ep, trash_row, with_counts):
  Dh = D // _NC
  G = Ep // (_NS * _KC)
  IT = G // 2
  rpt = Np // _NS

  mesh = plsc.VectorSubcoreMesh(core_axis_name="c", subcore_axis_name="s")
  if with_counts:
    out_type = (jax.ShapeDtypeStruct((_NC, Np, Dh), jnp.float32),
                jax.ShapeDtypeStruct((Np, _L), jnp.float32))
  else:
    out_type = jax.ShapeDtypeStruct((_NC, Np, Dh), jnp.float32)
  nz = rpt // _K
  assert rpt % _K == 0 and G % 2 == 0

  scratch = [
      pltpu.VMEM((G, _KC), jnp.int32),
      pltpu.VMEM((G, _KC), jnp.int32),
      pltpu.VMEM((2, _KC, Dh), jnp.float32),
      pltpu.VMEM_SHARED((Np, Dh), jnp.float32),
      pltpu.SemaphoreType.DMA,
      pltpu.SemaphoreType.DMA,
  ]
  if with_counts:
    scratch += [
        pltpu.VMEM((_KC, _L), jnp.float32),
        pltpu.VMEM((_K, _L), jnp.float32),
        pltpu.VMEM_SHARED((Np, _L), jnp.float32),
    ]

  def body(*refs):
    if with_counts:
      (table, src, dst, zacc, zcnt, ones, acc_out, cnt_out,
       src_a, dst_a, rows, sh_acc, gs0, gs1,
       ones_v, cnt_v, sh_cnt) = refs
    else:
      (table, src, dst, zacc, acc_out,
       src_a, dst_a, rows, sh_acc, gs0, gs1) = refs

    c = lax.axis_index("c")
    s = lax.axis_index("s")
    r0 = s * rpt

    pltpu.sync_copy(src.at[s], src_a)
    pltpu.sync_copy(dst.at[s], dst_a)

    def prep(i, carry):
      for j in range(_KC // _L):
        sl = pl.ds(j * _L, _L)
        sv = src_a[i, sl]
        dv = dst_a[i, sl]
        dst_a[i, sl] = jnp.where(
            sv == dv, jnp.full((_L,), trash_row, jnp.int32), dv)
        src_a[i, sl] = sv * 2 + c
      return carry

    lax.fori_loop(0, G, prep, 0)

    pltpu.sync_copy(zacc, rows.at[0, pl.ds(0, _K)])
    for z in range(nz):
      pltpu.sync_copy(rows.at[0, pl.ds(0, _K)],
                      sh_acc.at[pl.ds(r0 + z * _K, _K)])
    if with_counts:
      pltpu.sync_copy(ones, ones_v)

      @pl.when(c == 0)
      def _():
        pltpu.sync_copy(zcnt, cnt_v)
        for z in range(nz):
          pltpu.sync_copy(cnt_v, sh_cnt.at[pl.ds(r0 + z * _K, _K)])

    plsc.subcore_barrier()

    def g_wait(slot, sem):
      pltpu.make_async_copy(table.at[src_a.at[0]], rows.at[slot], sem).wait()

    def scatter(i, slot):
      pltpu.sync_copy(rows.at[slot], sh_acc.at[dst_a.at[i]], add=True)

    pltpu.async_copy(table.at[src_a.at[0]], rows.at[0], gs0)

    def step(i, carry):
      pltpu.async_copy(table.at[src_a.at[2 * i + 1]], rows.at[1], gs1)
      g_wait(0, gs0)
      scatter(2 * i, 0)
      if with_counts:
        @pl.when(c == 0)
        def _():
          pltpu.sync_copy(ones_v, sh_cnt.at[dst_a.at[2 * i]], add=True)

      @pl.when(i + 1 < IT)
      def _():
        pltpu.async_copy(table.at[src_a.at[2 * i + 2]], rows.at[0], gs0)

      g_wait(1, gs1)
      scatter(2 * i + 1, 1)
      if with_counts:
        @pl.when(c == 0)
        def _():
          pltpu.sync_copy(ones_v, sh_cnt.at[dst_a.at[2 * i + 1]], add=True)
      return carry

    lax.fori_loop(0, IT, step, 0)
    plsc.subcore_barrier()

    for z in range(nz):
      pltpu.sync_copy(sh_acc.at[pl.ds(r0 + z * _K, _K)], rows.at[0, pl.ds(0, _K)])
      pltpu.sync_copy(rows.at[0, pl.ds(0, _K)],
                      acc_out.at[c, pl.ds(r0 + z * _K, _K)])
    if with_counts:
      @pl.when(c == 0)
      def _():
        for z in range(nz):
          pltpu.sync_copy(sh_cnt.at[pl.ds(r0 + z * _K, _K)], cnt_v)
          pltpu.sync_copy(cnt_v, cnt_out.at[pl.ds(r0 + z * _K, _K)])

  return pl.kernel(
      body, out_type=out_type, mesh=mesh, scratch_types=scratch,
      compiler_params=pltpu.CompilerParams(use_tc_tiling_on_sc=False))


def _mm_relu_body(x_ref, w_ref, b_ref, o_ref):
  o_ref[...] = jnp.maximum(
      jnp.dot(x_ref[...], w_ref[...], preferred_element_type=jnp.float32)
      + b_ref[...], 0.0)


def _layer0_body(a0, a1, c0, xin, wla, wlb, bl, wa, ba, h_ref, o_ref):
  cnt = jnp.maximum(c0[:, :1], 1.0)
  mean = jnp.concatenate([a0[...], a1[...]], axis=1) / cnt
  h = (jnp.dot(xin[...], wla[...], preferred_element_type=jnp.float32)
       + jnp.dot(mean, wlb[...], preferred_element_type=jnp.float32)
       + bl[...])
  h = jnp.maximum(h, 0.0)
  nrm = jnp.sqrt(jnp.sum(h * h, axis=1, keepdims=True))
  h = h / jnp.maximum(nrm, 1e-12)
  h_ref[...] = h
  o_ref[...] = jnp.maximum(
      jnp.dot(h, wa[...], preferred_element_type=jnp.float32) + ba[...], 0.0)


def _layer1_body(a0, a1, c0, hin, wla, wlb, bl, wp1, bp1, wp2, bp2, o_ref):
  cnt = jnp.maximum(c0[:, :1], 1.0)
  mean = jnp.concatenate([a0[...], a1[...]], axis=1) / cnt
  h = (jnp.dot(hin[...], wla[...], preferred_element_type=jnp.float32)
       + jnp.dot(mean, wlb[...], preferred_element_type=jnp.float32)
       + bl[...])
  h = jnp.maximum(h, 0.0)
  nrm = jnp.sqrt(jnp.sum(h * h, axis=1, keepdims=True))
  h = h / jnp.maximum(nrm, 1e-12)
  p = (jnp.dot(jnp.dot(h, wp1[...], preferred_element_type=jnp.float32)
               + bp1[...], wp2[...], preferred_element_type=jnp.float32)
       + bp2[...])
  m = jnp.max(p, axis=1, keepdims=True)
  sh = p - m
  o_ref[...] = sh - jnp.log(jnp.sum(jnp.exp(sh), axis=1, keepdims=True))


def _row_spec(d):
  return pl.BlockSpec((_BR, d), lambda i: (i, 0))


def _full_spec(r, d):
  return pl.BlockSpec((r, d), lambda i: (0, 0))


def kernel(x, edge_index, batch, Wa0, ba0, Wl0, bl0, Wa1, ba1, Wl1, bl1,
           Wp1, bp1, Wp2, bp2):
  del batch
  N, D = x.shape
  E = edge_index.shape[1]
  H = Wl0.shape[1]
  O = Wp2.shape[1]
  f32 = jnp.float32

  Np = -(-(N + 1) // _BR) * _BR
  G = 2 * -(-E // (2 * _NS * _KC))
  Ep = G * _NS * _KC
  grid = (Np // _BR,)
  Dh = D // _NC

  x_p = jnp.pad(x, ((0, Np - N), (0, 0)))
  src_p = jnp.pad(edge_index[0], (0, Ep - E)).reshape(_NS, G, _KC)
  dst_p = jnp.pad(edge_index[1], (0, Ep - E)).reshape(_NS, G, _KC)
  zacc = jnp.zeros((_K, Dh), f32)
  zcnt = jnp.zeros((_K, _L), f32)
  ones = jnp.ones((_KC, _L), f32)

  out0 = pl.pallas_call(
      _mm_relu_body,
      grid=grid,
      in_specs=[_row_spec(D), _full_spec(D, D), _full_spec(1, D)],
      out_specs=_row_spec(D),
      out_shape=jax.ShapeDtypeStruct((Np, D), f32),
  )(x_p, Wa0, ba0.reshape(1, D))

  acc0, cnt = _sc_aggregate(Np, D, Ep, N, True)(
      out0.reshape(_NC * Np, Dh), src_p, dst_p, zacc, zcnt, ones)

  h0, out1 = pl.pallas_call(
      _layer0_body,
      grid=grid,
      in_specs=[_row_spec(Dh), _row_spec(Dh), _row_spec(_L),
                _row_spec(D), _full_spec(D, H), _full_spec(D, H),
                _full_spec(1, H), _full_spec(H, H), _full_spec(1, H)],
      out_specs=[_row_spec(H), _row_spec(H)],
      out_shape=[jax.ShapeDtypeStruct((Np, H), f32),
                 jax.ShapeDtypeStruct((Np, H), f32)],
  )(acc0[0], acc0[1], cnt, x_p, Wl0[:D], Wl0[D:],
    bl0.reshape(1, H), Wa1, ba1.reshape(1, H))

  acc1 = _sc_aggregate(Np, H, Ep, N, False)(
      out1.reshape(_NC * Np, Dh), src_p, dst_p, zacc)

  res = pl.pallas_call(
      _layer1_body,
      grid=grid,
      in_specs=[_row_spec(Dh), _row_spec(Dh), _row_spec(_L),
                _row_spec(H), _full_spec(H, H), _full_spec(H, H),
                _full_spec(1, H), _full_spec(H, H), _full_spec(1, H),
                _full_spec(H, O), _full_spec(1, O)],
      out_specs=_row_spec(O),
      out_shape=jax.ShapeDtypeStruct((Np, O), f32),
  )(acc1[0], acc1[1], cnt, h0, Wl1[:D], Wl1[D:],
    bl1.reshape(1, H), Wp1, bp1.reshape(1, H), Wp2, bp2.reshape(1, O))

  return res[:N]
